# baseline (speedup 1.0000x reference)
import jax
import jax.numpy as jnp
from jax.experimental import pallas as pl
from jax.experimental.pallas import tpu as pltpu


def _fused_kernel(x_ref, w_ref, adj_ref, bias_ref, out_ref, h_ref):
    @pl.when(pl.program_id(0) == 0)
    def _():
        x = x_ref[...].astype(jnp.bfloat16)
        w = w_ref[...].astype(jnp.bfloat16)
        h = jnp.dot(x, w, preferred_element_type=jnp.float32)
        h_ref[...] = h.astype(jnp.bfloat16)

    a = adj_ref[...].astype(jnp.bfloat16)
    acc = jnp.dot(a, h_ref[...], preferred_element_type=jnp.float32)
    out_ref[...] = acc + bias_ref[...]


def kernel(inputs, adj, Weight, Bias):
    n, d = inputs.shape
    bias2d = Bias.reshape(1, d)

    bm = 200
    if n % bm != 0:
        bm = n
    grid = (n // bm,)
    out = pl.pallas_call(
        _fused_kernel,
        grid=grid,
        in_specs=[
            pl.BlockSpec((n, d), lambda i: (0, 0)),
            pl.BlockSpec((d, d), lambda i: (0, 0)),
            pl.BlockSpec((bm, n), lambda i: (i, 0)),
            pl.BlockSpec((1, d), lambda i: (0, 0)),
        ],
        out_specs=pl.BlockSpec((bm, d), lambda i: (i, 0)),
        out_shape=jax.ShapeDtypeStruct((n, d), jnp.float32),
        scratch_shapes=[pltpu.VMEM((n, d), jnp.bfloat16)],
        compiler_params=pltpu.CompilerParams(
            dimension_semantics=("arbitrary",),
        ),
    )(inputs, Weight, adj, bias2d)
    return out


# speedup vs baseline: 1.0130x; 1.0130x over previous
import jax
import jax.numpy as jnp
from jax.experimental import pallas as pl
from jax.experimental.pallas import tpu as pltpu


def _fused_kernel(x_ref, w_ref, adj_ref, bias_ref, out_ref, h_ref):
    @pl.when(pl.program_id(0) == 0)
    def _():
        x = x_ref[...].astype(jnp.bfloat16)
        w = w_ref[...].astype(jnp.bfloat16)
        h = jnp.dot(x, w, preferred_element_type=jnp.float32)
        h_ref[...] = h.astype(jnp.bfloat16)

    a = adj_ref[...].astype(jnp.bfloat16)
    acc = jnp.dot(a, h_ref[...], preferred_element_type=jnp.float32)
    out_ref[...] = acc + bias_ref[...]


def kernel(inputs, adj, Weight, Bias):
    n, d = inputs.shape
    bias2d = Bias.reshape(1, d)

    bm = 400
    if n % bm != 0:
        bm = n
    grid = (n // bm,)
    out = pl.pallas_call(
        _fused_kernel,
        grid=grid,
        in_specs=[
            pl.BlockSpec((n, d), lambda i: (0, 0)),
            pl.BlockSpec((d, d), lambda i: (0, 0)),
            pl.BlockSpec((bm, n), lambda i: (i, 0)),
            pl.BlockSpec((1, d), lambda i: (0, 0)),
        ],
        out_specs=pl.BlockSpec((bm, d), lambda i: (i, 0)),
        out_shape=jax.ShapeDtypeStruct((n, d), jnp.float32),
        scratch_shapes=[pltpu.VMEM((n, d), jnp.bfloat16)],
        compiler_params=pltpu.CompilerParams(
            dimension_semantics=("arbitrary",),
            vmem_limit_bytes=63 * 1024 * 1024,
        ),
    )(inputs, Weight, adj, bias2d)
    return out


# speedup vs baseline: 1.0147x; 1.0016x over previous
import jax
import jax.numpy as jnp
from jax.experimental import pallas as pl
from jax.experimental.pallas import tpu as pltpu


def _fused_kernel(x_ref, w_ref, adj_ref, bias_ref, out_ref, h_ref):
    @pl.when(pl.program_id(0) == 0)
    def _():
        h_ref[...] = jnp.dot(
            x_ref[...], w_ref[...],
            precision=jax.lax.Precision.DEFAULT,
            preferred_element_type=jnp.float32,
        )

    acc = jnp.dot(
        adj_ref[...], h_ref[...],
        precision=jax.lax.Precision.DEFAULT,
        preferred_element_type=jnp.float32,
    )
    out_ref[...] = acc + bias_ref[...]


def kernel(inputs, adj, Weight, Bias):
    n, d = inputs.shape
    bias2d = Bias.reshape(1, d)

    bm = 400
    if n % bm != 0:
        bm = n
    grid = (n // bm,)
    out = pl.pallas_call(
        _fused_kernel,
        grid=grid,
        in_specs=[
            pl.BlockSpec((n, d), lambda i: (0, 0)),
            pl.BlockSpec((d, d), lambda i: (0, 0)),
            pl.BlockSpec((bm, n), lambda i: (i, 0)),
            pl.BlockSpec((1, d), lambda i: (0, 0)),
        ],
        out_specs=pl.BlockSpec((bm, d), lambda i: (i, 0)),
        out_shape=jax.ShapeDtypeStruct((n, d), jnp.float32),
        scratch_shapes=[pltpu.VMEM((n, d), jnp.float32)],
        compiler_params=pltpu.CompilerParams(
            dimension_semantics=("arbitrary",),
            vmem_limit_bytes=63 * 1024 * 1024,
        ),
    )(inputs, Weight, adj, bias2d)
    return out
